# baseline (speedup 1.0000x reference)
import functools

import jax
import jax.numpy as jnp
from jax import lax
from jax.experimental import pallas as pl
from jax.experimental.pallas import tpu as pltpu
from jax.experimental.pallas import tpu_sc as plsc

SEQ_LEN = 200
BATCH = 4096
EMB_DIM = 64
B = SEQ_LEN * BATCH
LANE = 128
NROWS = B // LANE
NC, NS = 2, 16
NW = NC * NS
ROWS_PER_W = NROWS // NW
K = 8
NCHUNK = ROWS_PER_W // K

_mesh = plsc.VectorSubcoreMesh(core_axis_name="c", subcore_axis_name="s")


@functools.partial(
    pl.kernel,
    mesh=_mesh,
    out_type=jax.ShapeDtypeStruct((NROWS, LANE, EMB_DIM), jnp.float32),
    scratch_types=[
        pltpu.VMEM((K, LANE), jnp.int32),
        pltpu.VMEM((K, LANE, EMB_DIM), jnp.float32),
        pltpu.SemaphoreType.DMA,
    ],
    compiler_params=pltpu.CompilerParams(use_tc_tiling_on_sc=False),
)
def _emb_lookup(words_hbm, table_hbm, out_hbm, idx_v, rows_v, sem):
    wid = lax.axis_index("s") * NC + lax.axis_index("c")
    base = wid * ROWS_PER_W

    def body(i, carry):
        row0 = base + i * K
        pltpu.sync_copy(words_hbm.at[pl.ds(row0, K)], idx_v)
        futs = [
            pltpu.async_copy(table_hbm.at[idx_v.at[j]], rows_v.at[j], sem)
            for j in range(K)
        ]
        for f in futs:
            f.wait()
        pltpu.sync_copy(rows_v, out_hbm.at[pl.ds(row0, K)])
        return carry

    lax.fori_loop(0, NCHUNK, body, 0)


def kernel(words, lut_weight):
    flat = words.astype(jnp.int32).reshape(NROWS, LANE)
    out = _emb_lookup(flat, lut_weight)
    return out.reshape(SEQ_LEN, BATCH, EMB_DIM)


# speedup vs baseline: 1.0146x; 1.0146x over previous
import functools

import jax
import jax.numpy as jnp
from jax import lax
from jax.experimental import pallas as pl
from jax.experimental.pallas import tpu as pltpu
from jax.experimental.pallas import tpu_sc as plsc

SEQ_LEN = 200
BATCH = 4096
EMB_DIM = 64
B = SEQ_LEN * BATCH
LANE = 128
NROWS = B // LANE
NC, NS = 2, 16
NW = NC * NS
ROWS_PER_W = NROWS // NW
K = 4
NCHUNK = ROWS_PER_W // K
NBUF = 2
NOUT = NCHUNK // NBUF

_mesh = plsc.VectorSubcoreMesh(core_axis_name="c", subcore_axis_name="s")


@functools.partial(
    pl.kernel,
    mesh=_mesh,
    out_type=jax.ShapeDtypeStruct((NROWS, LANE, EMB_DIM), jnp.float32),
    scratch_types=[
        pltpu.VMEM((NBUF, K, LANE), jnp.int32),
        pltpu.VMEM((NBUF, K, LANE, EMB_DIM), jnp.float32),
        pltpu.SemaphoreType.DMA((NBUF,)),
        pltpu.SemaphoreType.DMA((NBUF,)),
    ],
    compiler_params=pltpu.CompilerParams(use_tc_tiling_on_sc=False),
)
def _emb_lookup(words_hbm, table_hbm, out_hbm, idx_v, rows_v, gsem, wsem):
    wid = lax.axis_index("s") * NC + lax.axis_index("c")
    base = wid * ROWS_PER_W

    def body(t, carry):
        for b in range(NBUF):
            row0 = base + (t * NBUF + b) * K

            @pl.when(t > 0)
            def _():
                pltpu.make_async_copy(
                    rows_v.at[b], out_hbm.at[pl.ds(row0 - NBUF * K, K)],
                    wsem.at[b]).wait()

            pltpu.sync_copy(words_hbm.at[pl.ds(row0, K)], idx_v.at[b])
            for j in range(K):
                pltpu.async_copy(
                    table_hbm.at[idx_v.at[b, j]], rows_v.at[b, j], gsem.at[b])
        for b in range(NBUF):
            row0 = base + (t * NBUF + b) * K
            for j in range(K):
                pltpu.make_async_copy(
                    table_hbm.at[idx_v.at[b, j]], rows_v.at[b, j],
                    gsem.at[b]).wait()
            pltpu.async_copy(rows_v.at[b], out_hbm.at[pl.ds(row0, K)],
                             wsem.at[b])
        return carry

    lax.fori_loop(0, NOUT, body, 0)
    for b in range(NBUF):
        row0 = base + ((NOUT - 1) * NBUF + b) * K
        pltpu.make_async_copy(
            rows_v.at[b], out_hbm.at[pl.ds(row0, K)], wsem.at[b]).wait()


def kernel(words, lut_weight):
    flat = words.astype(jnp.int32).reshape(NROWS, LANE)
    out = _emb_lookup(flat, lut_weight)
    return out.reshape(SEQ_LEN, BATCH, EMB_DIM)
